# initial kernel scaffold (unmeasured)
import jax
import jax.numpy as jnp
from jax import lax
from jax.experimental import pallas as pl
from jax.experimental.pallas import tpu as pltpu

N_DEV = 4
HQ = 8
DH = 128
SQ = 1024
SKV = 1024
WIN = 128
SCALE = 0.08838834764831843
CHUNK = SQ // N_DEV
N_HOPS = 2 * (N_DEV - 1)


def kernel(x, Wq, K_ext, V_ext, Wo):

    def body(x_ref, wq_ref, k_hbm, v_hbm, wo_ref, out_ref,
             k_ref, v_ref, ctx_ref, comm_ref, send_sems, recv_sems,
             copy_sems):
        my = lax.axis_index("i")
        left = lax.rem(my + N_DEV - 1, N_DEV)
        right = lax.rem(my + 1, N_DEV)

        h0 = my * HQ
        k_copy = pltpu.make_async_copy(
            k_hbm.at[0, :, pl.ds(h0, HQ), :], k_ref, copy_sems.at[0])
        v_copy = pltpu.make_async_copy(
            v_hbm.at[0, :, pl.ds(h0, HQ), :], v_ref, copy_sems.at[1])
        k_copy.start()
        v_copy.start()

        barrier_sem = pltpu.get_barrier_semaphore()
        for nbr in (left, right):
            pl.semaphore_signal(
                barrier_sem, inc=1,
                device_id=(nbr,), device_id_type=pl.DeviceIdType.MESH)
        pl.semaphore_wait(barrier_sem, 2)

        xb = x_ref[0].astype(jnp.bfloat16)
        wqb = wq_ref[...].astype(jnp.bfloat16)
        q = jnp.dot(xb, wqb, preferred_element_type=jnp.float32)

        k_copy.wait()
        v_copy.wait()

        qi = lax.broadcasted_iota(jnp.int32, (SQ, SKV), 0)
        ki = lax.broadcasted_iota(jnp.int32, (SQ, SKV), 1)
        mask = jnp.abs(qi - ki) <= WIN

        for h in range(HQ):
            qh = q[:, h * DH:(h + 1) * DH].astype(jnp.bfloat16)
            kh = k_ref[:, h, :].astype(jnp.bfloat16)
            vh = v_ref[:, h, :].astype(jnp.bfloat16)
            scores = lax.dot_general(
                qh, kh, (((1,), (1,)), ((), ())),
                preferred_element_type=jnp.float32) * SCALE
            scores = jnp.where(mask, scores, -1e9)
            m = jnp.max(scores, axis=-1, keepdims=True)
            e = jnp.exp(scores - m)
            s = jnp.sum(e, axis=-1, keepdims=True)
            w = (e / s).astype(jnp.bfloat16)
            ctx_ref[:, h * DH:(h + 1) * DH] = jnp.dot(
                w, vh, preferred_element_type=jnp.float32)

        ctxb = ctx_ref[...].astype(jnp.bfloat16)
        wob = wo_ref[...].astype(jnp.bfloat16)
        out_ref[0] = jnp.dot(ctxb, wob, preferred_element_type=jnp.float32)

        def hop(hop_idx, send_idx, accumulate, recv_idx):
            rdma = pltpu.make_async_remote_copy(
                src_ref=out_ref.at[0, pl.ds(send_idx * CHUNK, CHUNK), :],
                dst_ref=comm_ref.at[hop_idx],
                send_sem=send_sems.at[hop_idx],
                recv_sem=recv_sems.at[hop_idx],
                device_id=(right,),
                device_id_type=pl.DeviceIdType.MESH,
            )
            rdma.start()
            rdma.wait()
            sl = pl.ds(recv_idx * CHUNK, CHUNK)
            if accumulate:
                out_ref[0, sl, :] = out_ref[0, sl, :] + comm_ref[hop_idx]
            else:
                out_ref[0, sl, :] = comm_ref[hop_idx]

        for st in range(N_DEV - 1):
            send_idx = lax.rem(my - st + N_DEV, N_DEV)
            recv_idx = lax.rem(my - st - 1 + N_DEV, N_DEV)
            hop(st, send_idx, True, recv_idx)
        for st in range(N_DEV - 1):
            send_idx = lax.rem(my + 1 - st + N_DEV, N_DEV)
            recv_idx = lax.rem(my - st + N_DEV, N_DEV)
            hop(N_DEV - 1 + st, send_idx, False, recv_idx)

    return pl.pallas_call(
        body,
        out_shape=jax.ShapeDtypeStruct((1, SQ, SQ), jnp.float32),
        in_specs=[
            pl.BlockSpec(memory_space=pltpu.VMEM),
            pl.BlockSpec(memory_space=pltpu.VMEM),
            pl.BlockSpec(memory_space=pltpu.ANY),
            pl.BlockSpec(memory_space=pltpu.ANY),
            pl.BlockSpec(memory_space=pltpu.VMEM),
        ],
        out_specs=pl.BlockSpec(memory_space=pltpu.VMEM),
        scratch_shapes=[
            pltpu.VMEM((SKV, HQ, DH), jnp.float32),
            pltpu.VMEM((SKV, HQ, DH), jnp.float32),
            pltpu.VMEM((SQ, HQ * DH), jnp.float32),
            pltpu.VMEM((N_HOPS, CHUNK, SQ), jnp.float32),
            pltpu.SemaphoreType.DMA((N_HOPS,)),
            pltpu.SemaphoreType.DMA((N_HOPS,)),
            pltpu.SemaphoreType.DMA((2,)),
        ],
        compiler_params=pltpu.CompilerParams(collective_id=0),
    )(x, Wq, K_ext, V_ext, Wo)


# baseline (device time: 112767 ns/iter reference)
import jax
import jax.numpy as jnp
from jax import lax
from jax.experimental import pallas as pl
from jax.experimental.pallas import tpu as pltpu

N_DEV = 4
HQ = 8
DH = 128
SQ = 1024
SKV = 1024
WIN = 128
SCALE = 0.08838834764831843
CHUNK = SQ // N_DEV
N_HOPS = 2 * (N_DEV - 1)


def kernel(x, Wq, K_ext, V_ext, Wo):

    def body(x_ref, wq_ref, k_hbm, v_hbm, wo_ref, out_ref,
             k_ref, v_ref, ctx_ref, comm_ref, send_sems, recv_sems,
             copy_sems):
        my = lax.axis_index("i")
        left = lax.rem(my + N_DEV - 1, N_DEV)
        right = lax.rem(my + 1, N_DEV)

        h0 = my * HQ
        k_copy = pltpu.make_async_copy(
            k_hbm.at[0, :, pl.ds(h0, HQ), :], k_ref, copy_sems.at[0])
        v_copy = pltpu.make_async_copy(
            v_hbm.at[0, :, pl.ds(h0, HQ), :], v_ref, copy_sems.at[1])
        k_copy.start()
        v_copy.start()

        barrier_sem = pltpu.get_barrier_semaphore()
        for nbr in (left, right):
            pl.semaphore_signal(
                barrier_sem, inc=1,
                device_id=(nbr,), device_id_type=pl.DeviceIdType.MESH)
        pl.semaphore_wait(barrier_sem, 2)

        xb = x_ref[0].astype(jnp.bfloat16)
        wqb = wq_ref[...].astype(jnp.bfloat16)
        q = jnp.dot(xb, wqb, preferred_element_type=jnp.float32)

        k_copy.wait()
        v_copy.wait()

        qi = lax.broadcasted_iota(jnp.int32, (SQ, SKV), 0)
        ki = lax.broadcasted_iota(jnp.int32, (SQ, SKV), 1)
        mask = jnp.abs(qi - ki) <= WIN

        for h in range(HQ):
            qh = q[:, h * DH:(h + 1) * DH].astype(jnp.bfloat16)
            kh = k_ref[:, h, :].astype(jnp.bfloat16)
            vh = v_ref[:, h, :].astype(jnp.bfloat16)
            scores = lax.dot_general(
                qh, kh, (((1,), (1,)), ((), ())),
                preferred_element_type=jnp.float32) * SCALE
            scores = jnp.where(mask, scores, -1e9)
            m = jnp.max(scores, axis=-1, keepdims=True)
            e = jnp.exp(scores - m)
            s = jnp.sum(e, axis=-1, keepdims=True)
            w = (e / s).astype(jnp.bfloat16)
            ctx_ref[:, h * DH:(h + 1) * DH] = jnp.dot(
                w, vh, preferred_element_type=jnp.float32)

        ctxb = ctx_ref[...].astype(jnp.bfloat16)
        wob = wo_ref[...].astype(jnp.bfloat16)
        out_ref[0] = jnp.dot(ctxb, wob, preferred_element_type=jnp.float32)

        def hop(hop_idx, send_idx, accumulate, recv_idx):
            rdma = pltpu.make_async_remote_copy(
                src_ref=out_ref.at[0, pl.ds(send_idx * CHUNK, CHUNK), :],
                dst_ref=comm_ref.at[hop_idx],
                send_sem=send_sems.at[hop_idx],
                recv_sem=recv_sems.at[hop_idx],
                device_id=(right,),
                device_id_type=pl.DeviceIdType.MESH,
            )
            rdma.start()
            rdma.wait()
            sl = pl.ds(recv_idx * CHUNK, CHUNK)
            if accumulate:
                out_ref[0, sl, :] = out_ref[0, sl, :] + comm_ref[hop_idx]
            else:
                out_ref[0, sl, :] = comm_ref[hop_idx]

        for st in range(N_DEV - 1):
            send_idx = lax.rem(my - st + N_DEV, N_DEV)
            recv_idx = lax.rem(my - st - 1 + N_DEV, N_DEV)
            hop(st, send_idx, True, recv_idx)
        for st in range(N_DEV - 1):
            send_idx = lax.rem(my + 1 - st + N_DEV, N_DEV)
            recv_idx = lax.rem(my - st + N_DEV, N_DEV)
            hop(N_DEV - 1 + st, send_idx, False, recv_idx)

    return pl.pallas_call(
        body,
        out_shape=jax.ShapeDtypeStruct((1, SQ, SQ), jnp.float32),
        in_specs=[
            pl.BlockSpec(memory_space=pltpu.VMEM),
            pl.BlockSpec(memory_space=pltpu.VMEM),
            pl.BlockSpec(memory_space=pltpu.MemorySpace.HBM),
            pl.BlockSpec(memory_space=pltpu.MemorySpace.HBM),
            pl.BlockSpec(memory_space=pltpu.VMEM),
        ],
        out_specs=pl.BlockSpec(memory_space=pltpu.VMEM),
        scratch_shapes=[
            pltpu.VMEM((SKV, HQ, DH), jnp.float32),
            pltpu.VMEM((SKV, HQ, DH), jnp.float32),
            pltpu.VMEM((SQ, HQ * DH), jnp.float32),
            pltpu.VMEM((N_HOPS, CHUNK, SQ), jnp.float32),
            pltpu.SemaphoreType.DMA((N_HOPS,)),
            pltpu.SemaphoreType.DMA((N_HOPS,)),
            pltpu.SemaphoreType.DMA((2,)),
        ],
        compiler_params=pltpu.CompilerParams(collective_id=0),
    )(x, Wq, K_ext, V_ext, Wo)


# device time: 62475 ns/iter; 1.8050x vs baseline; 1.8050x over previous
import jax
import jax.numpy as jnp
from jax import lax
from jax.experimental import pallas as pl
from jax.experimental.pallas import tpu as pltpu

N_DEV = 4
HQ = 8
DH = 128
SQ = 1024
SKV = 1024
WIN = 128
SCALE = 0.08838834764831843
HALF = SQ // 2
CH = HALF // N_DEV
N_HOPS = 2 * (N_DEV - 1)


def kernel(x, Wq, K_ext, V_ext, Wo):

    def body(x_ref, wq_ref, k_hbm, v_hbm, wo_ref, out_ref,
             k_ref, v_ref, ctx_ref, acc_ref, comm_r, comm_l,
             send_r, recv_r, send_l, recv_l, copy_sems):
        my = lax.axis_index("i")
        left = lax.rem(my + N_DEV - 1, N_DEV)
        right = lax.rem(my + 1, N_DEV)

        h0 = my * HQ
        k_copy = pltpu.make_async_copy(
            k_hbm.at[0, :, pl.ds(h0, HQ), :], k_ref, copy_sems.at[0])
        v_copy = pltpu.make_async_copy(
            v_hbm.at[0, :, pl.ds(h0, HQ), :], v_ref, copy_sems.at[1])
        k_copy.start()
        v_copy.start()

        barrier_sem = pltpu.get_barrier_semaphore()
        for nbr in (left, right):
            pl.semaphore_signal(
                barrier_sem, inc=1,
                device_id=(nbr,), device_id_type=pl.DeviceIdType.MESH)
        pl.semaphore_wait(barrier_sem, 2)

        xb = x_ref[0].astype(jnp.bfloat16)
        wqb = wq_ref[...].astype(jnp.bfloat16)
        q = jnp.dot(xb, wqb, preferred_element_type=jnp.float32)

        k_copy.wait()
        v_copy.wait()

        qi = lax.broadcasted_iota(jnp.int32, (SQ, SKV), 0)
        ki = lax.broadcasted_iota(jnp.int32, (SQ, SKV), 1)
        mask = jnp.abs(qi - ki) <= WIN

        for h in range(HQ):
            qh = q[:, h * DH:(h + 1) * DH].astype(jnp.bfloat16)
            kh = k_ref[:, h, :].astype(jnp.bfloat16)
            vh = v_ref[:, h, :].astype(jnp.bfloat16)
            scores = lax.dot_general(
                qh, kh, (((1,), (1,)), ((), ())),
                preferred_element_type=jnp.float32) * SCALE
            scores = jnp.where(mask, scores, -1e9)
            m = jnp.max(scores, axis=-1, keepdims=True)
            e = jnp.exp(scores - m)
            s = jnp.sum(e, axis=-1, keepdims=True)
            w = (e / s).astype(jnp.bfloat16)
            ctx_ref[:, h * DH:(h + 1) * DH] = jnp.dot(
                w, vh, preferred_element_type=jnp.float32)

        ctxb = ctx_ref[...].astype(jnp.bfloat16)
        wob = wo_ref[...].astype(jnp.bfloat16)
        acc_ref[...] = jnp.dot(
            ctxb, wob, preferred_element_type=jnp.float32
        ).astype(jnp.bfloat16)

        def r_off(idx):
            return lax.rem(idx + 2 * N_DEV, N_DEV) * CH

        def l_off(idx):
            return HALF + lax.rem(idx + 2 * N_DEV, N_DEV) * CH

        pending_sends = []

        def start_hop(hop, src_r, src_l):
            rdma_r = pltpu.make_async_remote_copy(
                src_ref=src_r, dst_ref=comm_r.at[hop],
                send_sem=send_r.at[hop], recv_sem=recv_r.at[hop],
                device_id=(right,), device_id_type=pl.DeviceIdType.MESH)
            rdma_l = pltpu.make_async_remote_copy(
                src_ref=src_l, dst_ref=comm_l.at[hop],
                send_sem=send_l.at[hop], recv_sem=recv_l.at[hop],
                device_id=(left,), device_id_type=pl.DeviceIdType.MESH)
            rdma_r.start()
            rdma_l.start()
            pending_sends.append(rdma_r)
            pending_sends.append(rdma_l)
            rdma_r.wait_recv()
            rdma_l.wait_recv()

        for st in range(N_DEV - 1):
            start_hop(
                st,
                acc_ref.at[pl.ds(r_off(my - st), CH), :],
                acc_ref.at[pl.ds(l_off(my + st), CH), :],
            )
            sl_r = pl.ds(r_off(my - st - 1), CH)
            acc_ref[sl_r, :] = acc_ref[sl_r, :] + comm_r[st]
            sl_l = pl.ds(l_off(my + st + 1), CH)
            acc_ref[sl_l, :] = acc_ref[sl_l, :] + comm_l[st]

        cr = r_off(my + 1)
        out_ref[0, pl.ds(cr, CH), :] = acc_ref[
            pl.ds(cr, CH), :].astype(jnp.float32)
        cl = l_off(my - 1)
        out_ref[0, pl.ds(cl, CH), :] = acc_ref[
            pl.ds(cl, CH), :].astype(jnp.float32)

        for st in range(N_DEV - 1):
            hop = N_DEV - 1 + st
            if st == 0:
                src_r = acc_ref.at[pl.ds(r_off(my + 1), CH), :]
                src_l = acc_ref.at[pl.ds(l_off(my - 1), CH), :]
            else:
                src_r = comm_r.at[hop - 1]
                src_l = comm_l.at[hop - 1]
            start_hop(hop, src_r, src_l)
            out_ref[0, pl.ds(r_off(my - st), CH), :] = comm_r[
                hop].astype(jnp.float32)
            out_ref[0, pl.ds(l_off(my + st), CH), :] = comm_l[
                hop].astype(jnp.float32)

        for rdma in pending_sends:
            rdma.wait_send()

    return pl.pallas_call(
        body,
        out_shape=jax.ShapeDtypeStruct((1, SQ, SQ), jnp.float32),
        in_specs=[
            pl.BlockSpec(memory_space=pltpu.VMEM),
            pl.BlockSpec(memory_space=pltpu.VMEM),
            pl.BlockSpec(memory_space=pltpu.MemorySpace.HBM),
            pl.BlockSpec(memory_space=pltpu.MemorySpace.HBM),
            pl.BlockSpec(memory_space=pltpu.VMEM),
        ],
        out_specs=pl.BlockSpec(memory_space=pltpu.VMEM),
        scratch_shapes=[
            pltpu.VMEM((SKV, HQ, DH), jnp.float32),
            pltpu.VMEM((SKV, HQ, DH), jnp.float32),
            pltpu.VMEM((SQ, HQ * DH), jnp.float32),
            pltpu.VMEM((SQ, SQ), jnp.bfloat16),
            pltpu.VMEM((N_HOPS, CH, SQ), jnp.bfloat16),
            pltpu.VMEM((N_HOPS, CH, SQ), jnp.bfloat16),
            pltpu.SemaphoreType.DMA((N_HOPS,)),
            pltpu.SemaphoreType.DMA((N_HOPS,)),
            pltpu.SemaphoreType.DMA((N_HOPS,)),
            pltpu.SemaphoreType.DMA((N_HOPS,)),
            pltpu.SemaphoreType.DMA((2,)),
        ],
        compiler_params=pltpu.CompilerParams(collective_id=0),
    )(x, Wq, K_ext, V_ext, Wo)
